# baseline (device time: 16357 ns/iter reference)
import jax
import jax.numpy as jnp
from jax import lax
from jax.experimental import pallas as pl
from jax.experimental.pallas import tpu as pltpu

N_DEV = 8
AXES = (1, 3, 4)
SCALE = 23.0

MSGS = (
    (1, 0, 0),
    (1, 0, 1),
    (1, 4, 1),
    (1, 4, 0),
    (3, 0, 0),
    (3, 0, 1),
    (3, 1, 0),
    (3, 1, 1),
    (4, 0, 1),
    (4, 0, 0),
    (4, 3, 0),
    (4, 3, 1),
    (6, 0, None),
)


def kernel(x, w_mat):
    m_per, k = x.shape
    _, n_per = w_mat.shape
    m_half = m_per // 2

    def body(x_ref, w_ref, out_ref, xg_ref, wb_ref, send_sems, recv_sems):
        my = lax.axis_index("i")

        xq = jnp.round(jnp.clip(x_ref[...] * SCALE, -127.0, 127.0))
        xg_ref[0] = xq.astype(jnp.int8)
        wb_ref[...] = w_ref[...].astype(jnp.bfloat16)

        barrier_sem = pltpu.get_barrier_semaphore()
        for L in AXES + (6,):
            pl.semaphore_signal(
                barrier_sem, inc=1,
                device_id=(my ^ L,), device_id_type=pl.DeviceIdType.MESH,
            )
        pl.semaphore_wait(barrier_sem, 4)

        def msg(j):
            L, s, h = MSGS[j]
            src = xg_ref.at[s] if h is None else xg_ref.at[s, h]
            dst = xg_ref.at[s ^ L] if h is None else xg_ref.at[s ^ L, h]
            return pltpu.make_async_remote_copy(
                src_ref=src,
                dst_ref=dst,
                send_sem=send_sems.at[j],
                recv_sem=recv_sems.at[j],
                device_id=(my ^ L,),
                device_id_type=pl.DeviceIdType.MESH,
            )

        m = [msg(j) for j in range(len(MSGS))]

        def gemm_half(slot, h):
            origin = my ^ slot
            chunk = xg_ref[slot, h].astype(jnp.bfloat16)
            y = jnp.dot(
                chunk, wb_ref[...], preferred_element_type=jnp.float32
            ) * (1.0 / SCALE)
            rows = pl.ds(origin * m_per + h * m_half, m_half)
            out_ref[rows, :] = y * jax.nn.sigmoid(y)

        for j in (12, 0, 1, 4, 5, 8, 9):
            m[j].start()
        gemm_half(0, 0)
        gemm_half(0, 1)

        m[0].wait_recv()
        m[6].start()
        gemm_half(1, 0)

        m[4].wait_recv()
        m[10].start()
        gemm_half(3, 0)

        m[8].wait_recv()
        m[2].start()
        gemm_half(4, 1)

        m[1].wait_recv()
        m[7].start()
        gemm_half(1, 1)

        m[5].wait_recv()
        m[11].start()
        gemm_half(3, 1)

        m[9].wait_recv()
        m[3].start()
        gemm_half(4, 0)

        m[2].wait_recv()
        gemm_half(5, 1)
        m[6].wait_recv()
        gemm_half(2, 0)
        m[10].wait_recv()
        gemm_half(7, 0)
        m[3].wait_recv()
        gemm_half(5, 0)
        m[7].wait_recv()
        gemm_half(2, 1)
        m[11].wait_recv()
        gemm_half(7, 1)

        m[12].wait_recv()
        gemm_half(6, 0)
        gemm_half(6, 1)

        for mm in m:
            mm.wait_send()


    return pl.pallas_call(
        body,
        out_shape=jax.ShapeDtypeStruct((N_DEV * m_per, n_per), jnp.float32),
        in_specs=[
            pl.BlockSpec(memory_space=pltpu.VMEM),
            pl.BlockSpec(memory_space=pltpu.VMEM),
        ],
        out_specs=pl.BlockSpec(memory_space=pltpu.VMEM),
        scratch_shapes=[
            pltpu.VMEM((N_DEV, 2, m_half, k), jnp.int8),
            pltpu.VMEM((k, n_per), jnp.bfloat16),
            pltpu.SemaphoreType.DMA((len(MSGS),)),
            pltpu.SemaphoreType.DMA((len(MSGS),)),
        ],
        compiler_params=pltpu.CompilerParams(collective_id=0),
    )(x.reshape(2, m_half, k), w_mat)


# device time: 15169 ns/iter; 1.0783x vs baseline; 1.0783x over previous
import jax
import jax.numpy as jnp
from jax import lax
from jax.experimental import pallas as pl
from jax.experimental.pallas import tpu as pltpu

N_DEV = 8
AXES = (1, 3, 4)
SCALE = 23.0

MSGS = (
    (1, 0, 0),
    (1, 0, 1),
    (1, 4, 1),
    (1, 4, 0),
    (1, 7, 0),
    (3, 0, 0),
    (3, 0, 1),
    (3, 1, 0),
    (3, 1, 1),
    (3, 5, 1),
    (4, 0, 1),
    (4, 0, 0),
    (4, 3, 0),
    (4, 3, 1),
)


def kernel(x, w_mat):
    m_per, k = x.shape
    _, n_per = w_mat.shape
    m_half = m_per // 2

    def body(x_ref, w_ref, out_ref, xg_ref, wb_ref, send_sems, recv_sems):
        my = lax.axis_index("i")

        xq = jnp.round(jnp.clip(x_ref[...] * SCALE, -127.0, 127.0))
        xg_ref[0] = xq.astype(jnp.int8)
        wb_ref[...] = w_ref[...].astype(jnp.bfloat16)

        barrier_sem = pltpu.get_barrier_semaphore()
        for L in AXES:
            pl.semaphore_signal(
                barrier_sem, inc=1,
                device_id=(my ^ L,), device_id_type=pl.DeviceIdType.MESH,
            )
        pl.semaphore_wait(barrier_sem, 3)

        def msg(j):
            L, s, h = MSGS[j]
            return pltpu.make_async_remote_copy(
                src_ref=xg_ref.at[s, h],
                dst_ref=xg_ref.at[s ^ L, h],
                send_sem=send_sems.at[j],
                recv_sem=recv_sems.at[j],
                device_id=(my ^ L,),
                device_id_type=pl.DeviceIdType.MESH,
            )

        m = [msg(j) for j in range(len(MSGS))]

        def gemm_half(slot, h):
            origin = my ^ slot
            chunk = xg_ref[slot, h].astype(jnp.bfloat16)
            y = jnp.dot(
                chunk, wb_ref[...], preferred_element_type=jnp.float32
            ) * (1.0 / SCALE)
            rows = pl.ds(origin * m_per + h * m_half, m_half)
            out_ref[rows, :] = y * jax.nn.sigmoid(y)

        for j in (0, 1, 5, 6, 10, 11):
            m[j].start()
        gemm_half(0, 0)
        gemm_half(0, 1)

        m[0].wait_recv()
        m[7].start()
        gemm_half(1, 0)

        m[5].wait_recv()
        m[12].start()
        gemm_half(3, 0)

        m[10].wait_recv()
        m[2].start()
        gemm_half(4, 1)

        m[1].wait_recv()
        m[8].start()
        gemm_half(1, 1)

        m[6].wait_recv()
        m[13].start()
        gemm_half(3, 1)

        m[11].wait_recv()
        m[3].start()
        gemm_half(4, 0)

        m[2].wait_recv()
        m[9].start()
        gemm_half(5, 1)

        m[7].wait_recv()
        gemm_half(2, 0)

        m[12].wait_recv()
        m[4].start()
        gemm_half(7, 0)

        m[3].wait_recv()
        gemm_half(5, 0)
        m[8].wait_recv()
        gemm_half(2, 1)
        m[13].wait_recv()
        gemm_half(7, 1)

        m[4].wait_recv()
        gemm_half(6, 0)
        m[9].wait_recv()
        gemm_half(6, 1)

        for mm in m:
            mm.wait_send()


    return pl.pallas_call(
        body,
        out_shape=jax.ShapeDtypeStruct((N_DEV * m_per, n_per), jnp.float32),
        in_specs=[
            pl.BlockSpec(memory_space=pltpu.VMEM),
            pl.BlockSpec(memory_space=pltpu.VMEM),
        ],
        out_specs=pl.BlockSpec(memory_space=pltpu.VMEM),
        scratch_shapes=[
            pltpu.VMEM((N_DEV, 2, m_half, k), jnp.int8),
            pltpu.VMEM((k, n_per), jnp.bfloat16),
            pltpu.SemaphoreType.DMA((len(MSGS),)),
            pltpu.SemaphoreType.DMA((len(MSGS),)),
        ],
        compiler_params=pltpu.CompilerParams(collective_id=0),
    )(x.reshape(2, m_half, k), w_mat)
